# baseline (speedup 1.0000x reference)
import functools

import jax
import jax.numpy as jnp
from jax import lax
from jax.experimental import pallas as pl
from jax.experimental.pallas import tpu as pltpu
from jax.experimental.pallas import tpu_sc as plsc

_S = 7
_NCH = 30
_BATCH = 128
_M = _BATCH * _S * _S
_NW = 32
_CPW = _M // _NW
_BPW = _BATCH // _NW
_NCHUNK = (_CPW + 15) // 16
_L_COORD = 5.0
_L_NOOBJ = 0.5


def _sqrt16(x):
    i = plsc.bitcast(x, jnp.int32)
    i = jnp.int32(0x1FBD1DF5) + jnp.right_shift(i, 1)
    y = plsc.bitcast(i, jnp.float32)
    for _ in range(3):
        y = 0.5 * (y + x / y)
    return y


def _corners(cx, cy, w, h):
    x = cx / float(_S)
    y = cy / float(_S)
    return x - 0.5 * w, y - 0.5 * h, x + 0.5 * w, y + 0.5 * h


def _sc_body(pred_hbm, tgt_hbm, out_hbm, pred_v, tgt_v, acc_v):
    cid = lax.axis_index("c")
    sid = lax.axis_index("s")
    wid = sid * 2 + cid
    pltpu.sync_copy(pred_hbm.at[pl.ds(wid * _BPW, _BPW)], pred_v)
    pltpu.sync_copy(tgt_hbm.at[pl.ds(wid * _BPW, _BPW)], tgt_v)

    lane = lax.iota(jnp.int32, 16)

    def chunk(j, acc):
        cells = j * 16 + lane
        valid = cells < _CPW
        cc = jnp.minimum(cells, _CPW - 1)
        b = cc // (_S * _S)
        r = cc - b * (_S * _S)
        i = r // _S
        jj = r - i * _S

        def gp(c):
            return plsc.load_gather(pred_v, [b, i, jj, lane * 0 + c])

        def gt(c):
            return plsc.load_gather(tgt_v, [b, i, jj, lane * 0 + c])

        t_x, t_y, t_w, t_h, t_conf = gt(0), gt(1), gt(2), gt(3), gt(4)
        tx1, ty1, tx2, ty2 = _corners(t_x, t_y, t_w, t_h)
        area_t = (tx2 - tx1) * (ty2 - ty1)

        def iou_of(px, py, pw, ph):
            x1, y1, x2, y2 = _corners(px, py, pw, ph)
            iw = jnp.maximum(jnp.minimum(x2, tx2) - jnp.maximum(x1, tx1), 0.0)
            ih = jnp.maximum(jnp.minimum(y2, ty2) - jnp.maximum(y1, ty1), 0.0)
            inter = iw * ih
            area_p = (x2 - x1) * (y2 - y1)
            return inter / (area_p + area_t - inter)

        p0 = [gp(c) for c in range(5)]
        p1 = [gp(c) for c in range(5, 10)]
        iou0 = iou_of(p0[0], p0[1], p0[2], p0[3])
        iou1 = iou_of(p1[0], p1[1], p1[2], p1[3])
        sel = iou1 > iou0
        max_iou = jnp.maximum(iou0, iou1)

        r = [jnp.where(sel, b1, b0) for b0, b1 in zip(p0, p1)]
        t1 = [gt(c) for c in range(5, 9)]
        tr = [jnp.where(sel, b1, b0)
              for b0, b1 in zip((t_x, t_y, t_w, t_h), t1)]

        dx = r[0] - tr[0]
        dy = r[1] - tr[1]
        l_xy = dx * dx + dy * dy
        dw = _sqrt16(r[2]) - _sqrt16(tr[2])
        dh = _sqrt16(r[3]) - _sqrt16(tr[3])
        l_wh = dw * dw + dh * dh
        do = r[4] - max_iou
        l_obj = do * do

        dn0 = p0[4] - t_conf
        dn1 = p1[4] - gt(9)
        l_noobj = dn0 * dn0 + dn1 * dn1

        l_cls = jnp.zeros((16,), jnp.float32)
        for c in range(10, 30):
            d = gp(c) - gt(c)
            l_cls = l_cls + d * d

        obj_f = jnp.where(valid & (t_conf > 0.0), 1.0, 0.0)
        noobj_f = jnp.where(valid & (t_conf == 0.0), 1.0, 0.0)
        cell = (obj_f * (_L_COORD * (l_xy + l_wh) + l_obj + l_cls)
                + _L_NOOBJ * noobj_f * l_noobj)
        return acc + cell

    acc = lax.fori_loop(0, _NCHUNK, chunk, jnp.zeros((16,), jnp.float32))
    acc_v[...] = acc
    pltpu.sync_copy(acc_v, out_hbm.at[wid])


@jax.jit
def kernel(pred_tensor, target_tensor):
    partials = pl.kernel(
        _sc_body,
        out_type=jax.ShapeDtypeStruct((_NW, 16), jnp.float32),
        mesh=plsc.VectorSubcoreMesh(core_axis_name="c", subcore_axis_name="s",
                                    num_cores=2, num_subcores=16),
        scratch_types=[
            pltpu.VMEM((_BPW, _S, _S, _NCH), jnp.float32),
            pltpu.VMEM((_BPW, _S, _S, _NCH), jnp.float32),
            pltpu.VMEM((16,), jnp.float32),
        ],
        compiler_params=pltpu.CompilerParams(needs_layout_passes=False,
                                             use_tc_tiling_on_sc=False),
    )(pred_tensor, target_tensor)
    return jnp.sum(partials) / float(_BATCH)


# speedup vs baseline: 1.8388x; 1.8388x over previous
import jax
import jax.numpy as jnp
from jax import lax
from jax.experimental import pallas as pl
from jax.experimental.pallas import tpu as pltpu
from jax.experimental.pallas import tpu_sc as plsc

_S = 7
_NCH = 30
_NCHP = 32
_BATCH = 128
_P = _S * _S
_NW = 16
_PPW = 4
_POS_F = _NCHP * _BATCH
_FPW = _PPW * _POS_F
_NG = _BATCH // 16
_L_COORD = 5.0
_L_NOOBJ = 0.5


def _sqrt16(x):
    i = plsc.bitcast(x, jnp.int32)
    i = jnp.int32(0x1FBD1DF5) + jnp.right_shift(i, 1)
    y = plsc.bitcast(i, jnp.float32)
    for _ in range(2):
        y = 0.5 * (y + x / y)
    return y


def _corners(cx, cy, w, h):
    x = cx / float(_S)
    y = cy / float(_S)
    return x - 0.5 * w, y - 0.5 * h, x + 0.5 * w, y + 0.5 * h


def _sc_body(pred_hbm, tgt_hbm, out_hbm, pred_v, tgt_v, acc_v,
             sem_p, sem_t, sem_p2, sem_t2):
    wid = lax.axis_index("s")
    k_lo = (_P * _NG * wid) // _NW
    k_hi = (_P * _NG * (wid + 1)) // _NW
    base_p = k_lo // _NG
    half = _PPW // 2 * _POS_F
    cp0 = pltpu.async_copy(
        pred_hbm.at[pl.ds(base_p * _POS_F, half)], pred_v.at[pl.ds(0, half)],
        sem_p)
    ct0 = pltpu.async_copy(
        tgt_hbm.at[pl.ds(base_p * _POS_F, half)], tgt_v.at[pl.ds(0, half)],
        sem_t)
    cp1 = pltpu.async_copy(
        pred_hbm.at[pl.ds(base_p * _POS_F + half, _FPW - half)],
        pred_v.at[pl.ds(half, _FPW - half)], sem_p2)
    ct1 = pltpu.async_copy(
        tgt_hbm.at[pl.ds(base_p * _POS_F + half, _FPW - half)],
        tgt_v.at[pl.ds(half, _FPW - half)], sem_t2)
    cp0.wait()
    ct0.wait()
    k_mid = jnp.clip((base_p + _PPW // 2) * _NG, k_lo, k_hi)

    def unit_loss(k, acc):
            p = k // _NG
            g = k - p * _NG
            off = (p - base_p) * _POS_F + g * 16

            def gp(c):
                return pred_v[pl.ds(off + c * _BATCH, 16)]

            def gt(c):
                return tgt_v[pl.ds(off + c * _BATCH, 16)]

            t_x, t_y, t_w, t_h, t_conf = gt(0), gt(1), gt(2), gt(3), gt(4)
            tx1, ty1, tx2, ty2 = _corners(t_x, t_y, t_w, t_h)
            area_t = (tx2 - tx1) * (ty2 - ty1)

            def iou_of(px, py, pw, ph):
                x1, y1, x2, y2 = _corners(px, py, pw, ph)
                iw = jnp.maximum(
                    jnp.minimum(x2, tx2) - jnp.maximum(x1, tx1), 0.0)
                ih = jnp.maximum(
                    jnp.minimum(y2, ty2) - jnp.maximum(y1, ty1), 0.0)
                inter = iw * ih
                area_p = (x2 - x1) * (y2 - y1)
                return inter / (area_p + area_t - inter)

            p0b = [gp(c) for c in range(5)]
            p1b = [gp(c) for c in range(5, 10)]
            iou0 = iou_of(p0b[0], p0b[1], p0b[2], p0b[3])
            iou1 = iou_of(p1b[0], p1b[1], p1b[2], p1b[3])
            sel = iou1 > iou0
            max_iou = jnp.maximum(iou0, iou1)

            r = [jnp.where(sel, b1, b0) for b0, b1 in zip(p0b, p1b)]
            t1 = [gt(c) for c in range(5, 9)]
            tr = [jnp.where(sel, b1, b0)
                  for b0, b1 in zip((t_x, t_y, t_w, t_h), t1)]

            dx = r[0] - tr[0]
            dy = r[1] - tr[1]
            l_xy = dx * dx + dy * dy
            l_wh = (r[2] + tr[2] - 2.0 * _sqrt16(r[2] * tr[2])
                    + r[3] + tr[3] - 2.0 * _sqrt16(r[3] * tr[3]))
            do = r[4] - max_iou
            l_obj = do * do

            dn0 = p0b[4] - t_conf
            dn1 = p1b[4] - gt(9)
            l_noobj = dn0 * dn0 + dn1 * dn1

            l_cls = jnp.zeros((16,), jnp.float32)
            for c in range(10, 30):
                d = gp(c) - gt(c)
                l_cls = l_cls + d * d

            obj_f = jnp.where(t_conf > 0.0, 1.0, 0.0)
            noobj_f = jnp.where(t_conf == 0.0, 1.0, 0.0)
            cell = (obj_f * (_L_COORD * (l_xy + l_wh) + l_obj + l_cls)
                    + _L_NOOBJ * noobj_f * l_noobj)
            return acc + cell

    acc = lax.fori_loop(k_lo, k_mid, unit_loss,
                        jnp.zeros((16,), jnp.float32))
    cp1.wait()
    ct1.wait()
    acc = lax.fori_loop(k_mid, k_hi, unit_loss, acc)
    acc_v[...] = acc
    pltpu.sync_copy(acc_v, out_hbm.at[wid])


@jax.jit
def kernel(pred_tensor, target_tensor):
    def prep(x):
        x = jnp.transpose(x, (1, 2, 3, 0)).reshape(_P, _NCH, _BATCH)
        z = jnp.zeros((_P, _NCHP - _NCH, _BATCH), jnp.float32)
        return jnp.concatenate([x, z], axis=1).reshape(_P * _POS_F)

    partials = pl.kernel(
        _sc_body,
        out_type=jax.ShapeDtypeStruct((_NW, 16), jnp.float32),
        mesh=plsc.VectorSubcoreMesh(core_axis_name="c", subcore_axis_name="s",
                                    num_cores=1, num_subcores=16),
        scratch_types=[
            pltpu.VMEM((_FPW,), jnp.float32),
            pltpu.VMEM((_FPW,), jnp.float32),
            pltpu.VMEM((16,), jnp.float32),
            pltpu.SemaphoreType.DMA,
            pltpu.SemaphoreType.DMA,
            pltpu.SemaphoreType.DMA,
            pltpu.SemaphoreType.DMA,
        ],
        compiler_params=pltpu.CompilerParams(needs_layout_passes=False),
    )(prep(pred_tensor), prep(target_tensor))
    return jnp.sum(partials) / float(_BATCH)


# speedup vs baseline: 1.8787x; 1.0217x over previous
import jax
import jax.numpy as jnp
from jax import lax
from jax.experimental import pallas as pl
from jax.experimental.pallas import tpu as pltpu
from jax.experimental.pallas import tpu_sc as plsc

_S = 7
_NCH = 30
_NCHP = 32
_BATCH = 128
_P = _S * _S
_NW = 16
_PPW = 4
_POS_F = _NCHP * _BATCH
_FPW = _PPW * _POS_F
_NG = _BATCH // 16
_L_COORD = 5.0
_L_NOOBJ = 0.5


def _sqrt16(x):
    i = plsc.bitcast(x, jnp.int32)
    i = jnp.int32(0x1FBD1DF5) + jnp.right_shift(i, 1)
    y = plsc.bitcast(i, jnp.float32)
    for _ in range(2):
        y = 0.5 * (y + x / y)
    return y


def _corners(cx, cy, w, h):
    x = cx / float(_S)
    y = cy / float(_S)
    return x - 0.5 * w, y - 0.5 * h, x + 0.5 * w, y + 0.5 * h


def _sc_body(pred_hbm, tgt_hbm, out_hbm, pred_v, tgt_v, acc_v, sem_p, sem_t):
    wid = lax.axis_index("s")
    k_lo = (_P * _NG * wid) // _NW
    k_hi = (_P * _NG * (wid + 1)) // _NW
    base_p = k_lo // _NG
    cp = pltpu.async_copy(
        pred_hbm.at[pl.ds(base_p * _POS_F, _FPW)], pred_v, sem_p)
    ct = pltpu.async_copy(
        tgt_hbm.at[pl.ds(base_p * _POS_F, _FPW)], tgt_v, sem_t)
    cp.wait()
    ct.wait()

    def unit_loss(k, acc):
            p = k // _NG
            g = k - p * _NG
            off = (p - base_p) * _POS_F + g * 16

            def gp(c):
                return pred_v[pl.ds(off + c * _BATCH, 16)]

            def gt(c):
                return tgt_v[pl.ds(off + c * _BATCH, 16)]

            t_x, t_y, t_w, t_h, t_conf = gt(0), gt(1), gt(2), gt(3), gt(4)
            tx1, ty1, tx2, ty2 = _corners(t_x, t_y, t_w, t_h)
            area_t = (tx2 - tx1) * (ty2 - ty1)

            def iou_of(px, py, pw, ph):
                x1, y1, x2, y2 = _corners(px, py, pw, ph)
                iw = jnp.maximum(
                    jnp.minimum(x2, tx2) - jnp.maximum(x1, tx1), 0.0)
                ih = jnp.maximum(
                    jnp.minimum(y2, ty2) - jnp.maximum(y1, ty1), 0.0)
                inter = iw * ih
                area_p = (x2 - x1) * (y2 - y1)
                return inter / (area_p + area_t - inter)

            p0b = [gp(c) for c in range(5)]
            p1b = [gp(c) for c in range(5, 10)]
            iou0 = iou_of(p0b[0], p0b[1], p0b[2], p0b[3])
            iou1 = iou_of(p1b[0], p1b[1], p1b[2], p1b[3])
            sel = iou1 > iou0
            max_iou = jnp.maximum(iou0, iou1)

            r = [jnp.where(sel, b1, b0) for b0, b1 in zip(p0b, p1b)]
            t1 = [gt(c) for c in range(5, 9)]
            tr = [jnp.where(sel, b1, b0)
                  for b0, b1 in zip((t_x, t_y, t_w, t_h), t1)]

            dx = r[0] - tr[0]
            dy = r[1] - tr[1]
            l_xy = dx * dx + dy * dy
            l_wh = (r[2] + tr[2] - 2.0 * _sqrt16(r[2] * tr[2])
                    + r[3] + tr[3] - 2.0 * _sqrt16(r[3] * tr[3]))
            do = r[4] - max_iou
            l_obj = do * do

            dn0 = p0b[4] - t_conf
            dn1 = p1b[4] - gt(9)
            l_noobj = dn0 * dn0 + dn1 * dn1

            l_cls = jnp.zeros((16,), jnp.float32)
            for c in range(10, 30):
                d = gp(c) - gt(c)
                l_cls = l_cls + d * d

            obj_f = jnp.where(t_conf > 0.0, 1.0, 0.0)
            noobj_f = jnp.where(t_conf == 0.0, 1.0, 0.0)
            cell = (obj_f * (_L_COORD * (l_xy + l_wh) + l_obj + l_cls)
                    + _L_NOOBJ * noobj_f * l_noobj)
            return acc + cell

    acc = lax.fori_loop(k_lo, k_hi, unit_loss,
                        jnp.zeros((16,), jnp.float32))
    acc_v[...] = acc
    pltpu.sync_copy(acc_v, out_hbm.at[wid])


@jax.jit
def kernel(pred_tensor, target_tensor):
    def prep(x):
        x = jnp.transpose(x, (1, 2, 3, 0)).reshape(_P, _NCH, _BATCH)
        z = jnp.zeros((_P, _NCHP - _NCH, _BATCH), jnp.float32)
        return jnp.concatenate([x, z], axis=1).reshape(_P * _POS_F)

    partials = pl.kernel(
        _sc_body,
        out_type=jax.ShapeDtypeStruct((_NW, 16), jnp.float32),
        mesh=plsc.VectorSubcoreMesh(core_axis_name="c", subcore_axis_name="s",
                                    num_cores=1, num_subcores=16),
        scratch_types=[
            pltpu.VMEM((_FPW,), jnp.float32),
            pltpu.VMEM((_FPW,), jnp.float32),
            pltpu.VMEM((16,), jnp.float32),
            pltpu.SemaphoreType.DMA,
            pltpu.SemaphoreType.DMA,
        ],
        compiler_params=pltpu.CompilerParams(needs_layout_passes=False),
    )(prep(pred_tensor), prep(target_tensor))
    return jnp.sum(partials) / float(_BATCH)


# speedup vs baseline: 1.9222x; 1.0231x over previous
import jax
import jax.numpy as jnp
from jax import lax
from jax.experimental import pallas as pl
from jax.experimental.pallas import tpu as pltpu
from jax.experimental.pallas import tpu_sc as plsc

_S = 7
_NCH = 30
_NCHP = 32
_BATCH = 128
_P = _S * _S
_NW = 16
_PPW = 4
_POS_F = _NCHP * _BATCH
_FPW = _PPW * _POS_F
_NG = _BATCH // 16
_L_COORD = 5.0
_L_NOOBJ = 0.5


def _sqrt16(x):
    i = plsc.bitcast(x, jnp.int32)
    i = jnp.int32(0x5F3759DF) - jnp.right_shift(i, 1)
    z = plsc.bitcast(i, jnp.float32)
    xh = 0.5 * x
    for _ in range(2):
        z = z * (1.5 - xh * z * z)
    return x * z


def _corners(cx, cy, w, h):
    x = cx * (1.0 / _S)
    y = cy * (1.0 / _S)
    return x - 0.5 * w, y - 0.5 * h, x + 0.5 * w, y + 0.5 * h


def _sc_body(pred_hbm, tgt_hbm, out_hbm, pred_v, tgt_v, acc_v, sem_p, sem_t):
    wid = lax.axis_index("s")
    k_lo = (_P * _NG * wid) // _NW
    k_hi = (_P * _NG * (wid + 1)) // _NW
    base_p = k_lo // _NG
    cp = pltpu.async_copy(
        pred_hbm.at[pl.ds(base_p * _POS_F, _FPW)], pred_v, sem_p)
    ct = pltpu.async_copy(
        tgt_hbm.at[pl.ds(base_p * _POS_F, _FPW)], tgt_v, sem_t)
    cp.wait()
    ct.wait()

    def unit_loss(k, acc):
            p = k // _NG
            g = k - p * _NG
            off = (p - base_p) * _POS_F + g * 16

            def gp(c):
                return pred_v[pl.ds(off + c * _BATCH, 16)]

            def gt(c):
                return tgt_v[pl.ds(off + c * _BATCH, 16)]

            t_x, t_y, t_w, t_h, t_conf = gt(0), gt(1), gt(2), gt(3), gt(4)
            tx1, ty1, tx2, ty2 = _corners(t_x, t_y, t_w, t_h)
            area_t = (tx2 - tx1) * (ty2 - ty1)

            def iou_parts(px, py, pw, ph):
                x1, y1, x2, y2 = _corners(px, py, pw, ph)
                iw = jnp.maximum(
                    jnp.minimum(x2, tx2) - jnp.maximum(x1, tx1), 0.0)
                ih = jnp.maximum(
                    jnp.minimum(y2, ty2) - jnp.maximum(y1, ty1), 0.0)
                inter = iw * ih
                area_p = (x2 - x1) * (y2 - y1)
                return inter, area_p + area_t - inter

            p0b = [gp(c) for c in range(5)]
            p1b = [gp(c) for c in range(5, 10)]
            in0, de0 = iou_parts(p0b[0], p0b[1], p0b[2], p0b[3])
            in1, de1 = iou_parts(p1b[0], p1b[1], p1b[2], p1b[3])
            sel = in1 * de0 > in0 * de1
            max_iou = jnp.where(sel, in1, in0) / jnp.where(sel, de1, de0)

            r = [jnp.where(sel, b1, b0) for b0, b1 in zip(p0b, p1b)]
            t1 = [gt(c) for c in range(5, 9)]
            tr = [jnp.where(sel, b1, b0)
                  for b0, b1 in zip((t_x, t_y, t_w, t_h), t1)]

            dx = r[0] - tr[0]
            dy = r[1] - tr[1]
            l_xy = dx * dx + dy * dy
            l_wh = (r[2] + tr[2] - 2.0 * _sqrt16(r[2] * tr[2])
                    + r[3] + tr[3] - 2.0 * _sqrt16(r[3] * tr[3]))
            do = r[4] - max_iou
            l_obj = do * do

            dn0 = p0b[4] - t_conf
            dn1 = p1b[4] - gt(9)
            l_noobj = dn0 * dn0 + dn1 * dn1

            l_cls = jnp.zeros((16,), jnp.float32)
            for c in range(10, 30):
                d = gp(c) - gt(c)
                l_cls = l_cls + d * d

            obj_f = jnp.where(t_conf > 0.0, 1.0, 0.0)
            noobj_f = jnp.where(t_conf == 0.0, 1.0, 0.0)
            cell = (obj_f * (_L_COORD * (l_xy + l_wh) + l_obj + l_cls)
                    + _L_NOOBJ * noobj_f * l_noobj)
            return acc + cell

    acc = lax.fori_loop(k_lo, k_hi, unit_loss,
                        jnp.zeros((16,), jnp.float32))
    acc_v[...] = acc
    pltpu.sync_copy(acc_v, out_hbm.at[wid])


@jax.jit
def kernel(pred_tensor, target_tensor):
    def prep(x):
        x = jnp.transpose(x, (1, 2, 3, 0)).reshape(_P, _NCH, _BATCH)
        z = jnp.zeros((_P, _NCHP - _NCH, _BATCH), jnp.float32)
        return jnp.concatenate([x, z], axis=1).reshape(_P * _POS_F)

    partials = pl.kernel(
        _sc_body,
        out_type=jax.ShapeDtypeStruct((_NW, 16), jnp.float32),
        mesh=plsc.VectorSubcoreMesh(core_axis_name="c", subcore_axis_name="s",
                                    num_cores=1, num_subcores=16),
        scratch_types=[
            pltpu.VMEM((_FPW,), jnp.float32),
            pltpu.VMEM((_FPW,), jnp.float32),
            pltpu.VMEM((16,), jnp.float32),
            pltpu.SemaphoreType.DMA,
            pltpu.SemaphoreType.DMA,
        ],
        compiler_params=pltpu.CompilerParams(needs_layout_passes=False),
    )(prep(pred_tensor), prep(target_tensor))
    return jnp.sum(partials) / float(_BATCH)
